# baseline (speedup 1.0000x reference)
import functools

import jax
import jax.numpy as jnp
from jax import lax
from jax.experimental import pallas as pl
from jax.experimental.pallas import tpu as pltpu
from jax.experimental.pallas import tpu_sc as plsc

S, B, D = 2048, 2, 1024
E, K, C = 64, 2, 256
F_EXP = 512
F_FFN = 2048
T = S * B
KT = K * T
EC = E * C
Y_ROWS = EC + 256
TILE = 256
N_TILES = T // TILE

_f32 = jnp.float32
_i32 = jnp.int32


def _stage1_body(x_ref, res_ref, fw1_ref, fb1_ref, fw2_ref, fb2_ref,
                 lng_ref, lnb_ref, rw_ref,
                 xr_ref, xn_ref, meta_ref, stats_ref):
    i = pl.program_id(0)
    xt = x_ref[...]
    h = jnp.maximum(
        jnp.dot(xt, fw1_ref[...], preferred_element_type=_f32) + fb1_ref[...],
        0.0)
    ff = jnp.dot(h, fw2_ref[...], preferred_element_type=_f32) + fb2_ref[...]
    res_t = jnp.swapaxes(res_ref[...], 0, 1).reshape(TILE, D)
    xr = ff + res_t
    xr_ref[...] = xr
    m = jnp.mean(xr, axis=1, keepdims=True)
    d = xr - m
    v = jnp.mean(d * d, axis=1, keepdims=True)
    xn = d / jnp.sqrt(v + 1e-5) * lng_ref[...] + lnb_ref[...]
    xn_ref[...] = xn
    logits = jnp.dot(xn, rw_ref[...], preferred_element_type=_f32)
    mx = jnp.max(logits, axis=1, keepdims=True)
    ex = jnp.exp(logits - mx)
    p = ex / jnp.sum(ex, axis=1, keepdims=True)
    io = lax.broadcasted_iota(_i32, (TILE, E), 1)
    m0 = jnp.max(p, axis=1, keepdims=True)
    i0 = jnp.min(jnp.where(p == m0, io, E), axis=1, keepdims=True)
    mask0 = io == i0
    p2 = jnp.where(mask0, -1.0, p)
    m1 = jnp.max(p2, axis=1, keepdims=True)
    i1 = jnp.min(jnp.where(p2 == m1, io, E), axis=1, keepdims=True)
    den = m0 + m1 + 1e-9
    g0 = m0 / den
    g1 = m1 / den
    meta_ref[...] = jnp.concatenate(
        [i0.astype(_f32), i1.astype(_f32), g0, g1,
         jnp.zeros((TILE, 4), _f32)], axis=1)
    srow = jnp.concatenate(
        [jnp.sum(p, axis=0, keepdims=True),
         jnp.sum(mask0.astype(_f32), axis=0, keepdims=True)], axis=1)

    @pl.when(i == 0)
    def _():
        stats_ref[...] = jnp.zeros_like(stats_ref)

    stats_ref[...] += srow


def _stage1(x2d, res_bsd, ffn_w1, ffn_b1, ffn_w2, ffn_b2, ln_g, ln_b, router_w):
    tok = lambda i: (i, 0)
    fixed = lambda i: (0, 0)
    return pl.pallas_call(
        _stage1_body,
        grid=(N_TILES,),
        in_specs=[
            pl.BlockSpec((TILE, D), tok),
            pl.BlockSpec((B, TILE // B, D), lambda i: (0, i, 0)),
            pl.BlockSpec((D, F_FFN), fixed),
            pl.BlockSpec((1, F_FFN), fixed),
            pl.BlockSpec((F_FFN, D), fixed),
            pl.BlockSpec((1, D), fixed),
            pl.BlockSpec((1, D), fixed),
            pl.BlockSpec((1, D), fixed),
            pl.BlockSpec((D, E), fixed),
        ],
        out_specs=[
            pl.BlockSpec((TILE, D), tok),
            pl.BlockSpec((TILE, D), tok),
            pl.BlockSpec((TILE, 8), tok),
            pl.BlockSpec((1, 2 * E), fixed),
        ],
        out_shape=[
            jax.ShapeDtypeStruct((T, D), _f32),
            jax.ShapeDtypeStruct((T, D), _f32),
            jax.ShapeDtypeStruct((T, 8), _f32),
            jax.ShapeDtypeStruct((1, 2 * E), _f32),
        ],
    )(x2d, res_bsd, ffn_w1, ffn_b1, ffn_w2, ffn_b2, ln_g, ln_b, router_w)


_RB = 512
_NRB = KT // _RB


def _stage2_body(meta_ref, stats_ref,
                 slot_tok_ref, gate_t_ref, sl_ref, laux_ref,
                 m1_acc, m1l_acc, m2_acc):
    ior = lax.broadcasted_iota(_i32, (_RB, _RB), 0)
    ioc = lax.broadcasted_iota(_i32, (_RB, _RB), 1)
    tri = (ior >= ioc).astype(_f32)
    ioE = lax.broadcasted_iota(_i32, (_RB, E), 1).astype(_f32)
    ioC = lax.broadcasted_iota(_i32, (_RB, C), 1).astype(_f32)
    m1_acc[...] = jnp.zeros_like(m1_acc)
    m1l_acc[...] = jnp.zeros_like(m1l_acc)
    m2_acc[...] = jnp.zeros_like(m2_acc)

    def body(blk, offset):
        k = blk // (_NRB // 2)
        rowoff = (blk % (_NRB // 2)) * _RB
        mb = meta_ref[pl.ds(rowoff, _RB), :]
        e_col = jnp.where(k == 0, mb[:, 0:1], mb[:, 1:2])
        g_col = jnp.where(k == 0, mb[:, 2:3], mb[:, 3:4])
        ohb = (e_col == ioE).astype(_f32)
        cin = jnp.dot(tri, ohb, preferred_element_type=_f32) + offset
        pos = jnp.sum(cin * ohb, axis=1, keepdims=True) - 1.0
        keep = pos < C
        pohb = (pos == ioC).astype(_f32)
        tv = lax.broadcasted_iota(_i32, (_RB, 1), 0).astype(_f32) + rowoff
        thi = jnp.floor(tv * (1.0 / 64.0))
        tlo = tv - thi * 64.0
        cdim = (((0,), (0,)), ((), ()))
        m1_acc[...] += lax.dot_general(ohb * (thi + 1.0), pohb, cdim,
                                       preferred_element_type=_f32)
        m1l_acc[...] += lax.dot_general(ohb * (tlo + 1.0), pohb, cdim,
                                        preferred_element_type=_f32)
        m2_acc[...] += lax.dot_general(pohb, ohb * g_col, cdim,
                                       preferred_element_type=_f32)
        slf = jnp.where(keep, e_col * C + pos, float(EC))
        sl_ref[pl.ds(blk * _RB, _RB), :] = slf.astype(_i32)
        return cin[_RB - 1:_RB, :]

    lax.fori_loop(0, _NRB, body, jnp.zeros((1, E), _f32))

    occ = m1_acc[...] >= 0.5
    tok = (m1_acc[...] - 1.0) * 64.0 + (m1l_acc[...] - 1.0)
    fb_e = lax.broadcasted_iota(_i32, (E, C), 0)
    fb_c = lax.broadcasted_iota(_i32, (E, C), 1)
    fallback = (fb_e * C + fb_c) & (T - 1)
    slot_tok_ref[...] = jnp.where(occ, tok.astype(_i32), fallback)
    gate_t_ref[...] = m2_acc[...]
    me = stats_ref[0:1, 0:E] * (1.0 / T)
    ce = stats_ref[0:1, E:2 * E] * (1.0 / T)
    laux_ref[...] = jnp.sum(me * ce, axis=1, keepdims=True) * float(E)


def _stage2(meta, stats):
    return pl.pallas_call(
        _stage2_body,
        out_shape=[
            jax.ShapeDtypeStruct((E, C), _i32),
            jax.ShapeDtypeStruct((C, E), _f32),
            jax.ShapeDtypeStruct((KT, 1), _i32),
            jax.ShapeDtypeStruct((1, 1), _f32),
        ],
        scratch_shapes=[
            pltpu.VMEM((E, C), _f32),
            pltpu.VMEM((E, C), _f32),
            pltpu.VMEM((C, E), _f32),
        ],
    )(meta, stats)


_NC, _NS = 2, 16
_NW = _NC * _NS
_CH = 32


def _sc_gather_DEBUG_XLA(table, idx):
    return jnp.take(table, idx, axis=0)


def _sc_gather(table, idx):
    n, d = table.shape
    r = idx.shape[0]
    rpw = r // _NW
    nch = rpw // _CH
    mesh = plsc.VectorSubcoreMesh(core_axis_name="c", subcore_axis_name="s")

    dt = table.dtype

    def body(table_ref, idx_ref, out_ref, idx_v, buf0, buf1, sem0, sem1):
        wid = lax.axis_index("s") * _NC + lax.axis_index("c")
        base = wid * rpw
        pltpu.sync_copy(idx_ref.at[pl.ds(base, rpw)], idx_v)
        bufs = (buf0, buf1)
        sems = (sem0, sem1)
        cps = [None, None]
        for j in range(nch):
            bsel = j & 1
            cps[bsel] = pltpu.async_copy(
                table_ref.at[idx_v.at[pl.ds(j * _CH, _CH)]], bufs[bsel],
                sems[bsel])
            if j >= 1:
                cps[1 - bsel].wait()
                pltpu.sync_copy(bufs[1 - bsel],
                                out_ref.at[pl.ds(base + (j - 1) * _CH, _CH)])
        last = (nch - 1) & 1
        cps[last].wait()
        pltpu.sync_copy(bufs[last],
                        out_ref.at[pl.ds(base + (nch - 1) * _CH, _CH)])

    return pl.kernel(
        body,
        out_type=jax.ShapeDtypeStruct((r, d), dt),
        mesh=mesh,
        scratch_types=[
            pltpu.VMEM((rpw,), _i32),
            pltpu.VMEM((_CH, d), dt),
            pltpu.VMEM((_CH, d), dt),
            pltpu.SemaphoreType.DMA,
            pltpu.SemaphoreType.DMA,
        ],
    )(table, idx)


_EH = E // 2


def _expert_block(buf_ref, w1_ref, b1_ref, w2_ref, b2_ref, gate_ref, eg):
    xb = buf_ref[...].astype(jnp.bfloat16)
    w1b = w1_ref[0].astype(jnp.bfloat16)
    h = jnp.maximum(
        jnp.dot(xb, w1b, preferred_element_type=_f32) + b1_ref[0], 0.0)
    w2b = w2_ref[0].astype(jnp.bfloat16)
    y = jnp.dot(h.astype(jnp.bfloat16), w2b,
                preferred_element_type=_f32) + b2_ref[0]
    ioe = lax.broadcasted_iota(_i32, (C, E), 1)
    gcol = jnp.sum(jnp.where(ioe == eg, gate_ref[...], 0.0),
                   axis=1, keepdims=True)
    return y * gcol


def _stage4a_body(buf_ref, w1_ref, b1_ref, w2_ref, b2_ref, gate_ref, y_ref):
    e = pl.program_id(0)
    y_ref[...] = _expert_block(buf_ref, w1_ref, b1_ref, w2_ref, b2_ref,
                               gate_ref, e)


def _stage4b_body(yprev_ref, buf_ref, w1_ref, b1_ref, w2_ref, b2_ref,
                  gate_ref, y_ref):
    del yprev_ref
    e = pl.program_id(0)

    @pl.when(e < _EH)
    def _():
        y_ref[...] = _expert_block(buf_ref, w1_ref, b1_ref, w2_ref, b2_ref,
                                   gate_ref, e + _EH)

    @pl.when(e == _EH)
    def _():
        y_ref[...] = jnp.zeros_like(y_ref)


def _expert_specs(off, clamp):
    em = lambda e: (jnp.minimum(e + off, clamp), 0)
    em3 = lambda e: (jnp.minimum(e + off, clamp), 0, 0)
    fixed = lambda e: (0, 0)
    return [
        pl.BlockSpec((C, D), lambda e: (jnp.minimum(e, _EH - 1), 0)),
        pl.BlockSpec((1, D, F_EXP), em3),
        pl.BlockSpec((1, 1, F_EXP), em3),
        pl.BlockSpec((1, F_EXP, D), em3),
        pl.BlockSpec((1, 1, D), em3),
        pl.BlockSpec((C, E), fixed),
    ]


def _stage4(buf_a, buf_b, w1, b1, w2, b2, gate_t):
    b1r = b1.reshape(E, 1, F_EXP)
    b2r = b2.reshape(E, 1, D)
    y1 = pl.pallas_call(
        _stage4a_body,
        grid=(_EH,),
        in_specs=_expert_specs(0, E - 1),
        out_specs=pl.BlockSpec((C, D), lambda e: (e, 0)),
        out_shape=jax.ShapeDtypeStruct((Y_ROWS, D), _f32),
    )(buf_a, w1, b1r, w2, b2r, gate_t)
    return pl.pallas_call(
        _stage4b_body,
        grid=(_EH + 1,),
        in_specs=[pl.BlockSpec(memory_space=pl.ANY)]
        + _expert_specs(_EH, E - 1),
        out_specs=pl.BlockSpec((C, D), lambda e: (e + _EH, 0)),
        out_shape=jax.ShapeDtypeStruct((Y_ROWS, D), _f32),
        input_output_aliases={0: 0},
    )(y1, buf_b, w1, b1r, w2, b2r, gate_t)


def _combine1_body(ya_ref, yb_ref, xr_ref, out_ref):
    out_ref[...] = ya_ref[...] + yb_ref[...] + xr_ref[...]


def _combine2_body(prev_ref, ya_ref, yb_ref, xr_ref, out_ref):
    del prev_ref
    out_ref[...] = ya_ref[...] + yb_ref[...] + xr_ref[...]


_HT = N_TILES // 2


def _combine(ga0, ga1, gb0, gb1, xr):
    tokh = lambda i: (i, 0)
    out1 = pl.pallas_call(
        _combine1_body,
        grid=(_HT,),
        in_specs=[
            pl.BlockSpec((TILE, D), tokh),
            pl.BlockSpec((TILE, D), tokh),
            pl.BlockSpec((TILE, D), tokh),
        ],
        out_specs=pl.BlockSpec((TILE, D), tokh),
        out_shape=jax.ShapeDtypeStruct((T, D), _f32),
    )(ga0, ga1, xr)
    return pl.pallas_call(
        _combine2_body,
        grid=(_HT,),
        in_specs=[
            pl.BlockSpec(memory_space=pl.ANY),
            pl.BlockSpec((TILE, D), tokh),
            pl.BlockSpec((TILE, D), tokh),
            pl.BlockSpec((TILE, D), lambda i: (i + _HT, 0)),
        ],
        out_specs=pl.BlockSpec((TILE, D), lambda i: (i + _HT, 0)),
        out_shape=jax.ShapeDtypeStruct((T, D), _f32),
        input_output_aliases={0: 0},
    )(out1, gb0, gb1, xr)


def kernel(x, residual, router_w, w1, b1, w2, b2,
           ffn_w1, ffn_b1, ffn_w2, ffn_b2, ln_g, ln_b):
    x2d = x.reshape(T, D)
    xr, xn, meta, stats = _stage1(
        x2d, residual, ffn_w1, ffn_b1.reshape(1, F_FFN), ffn_w2,
        ffn_b2.reshape(1, D), ln_g.reshape(1, D), ln_b.reshape(1, D),
        router_w)
    slot_tok, gate_t, sl, laux = _stage2(meta, stats)
    slot_flat = slot_tok.reshape(-1)
    buf_a = _sc_gather(xn, slot_flat[:EC // 2])
    buf_b = _sc_gather(xn, slot_flat[EC // 2:])
    y = _stage4(buf_a, buf_b, w1, b1, w2, b2, gate_t)
    sl_flat = sl.reshape(-1)
    half = T // 2
    ga0 = _sc_gather(y, sl_flat[:half])
    ga1 = _sc_gather(y, sl_flat[T:T + half])
    gb0 = _sc_gather(y, sl_flat[half:T])
    gb1 = _sc_gather(y, sl_flat[T + half:])
    out2d = _combine(ga0, ga1, gb0, gb1, xr)
    return out2d.reshape(S, B, D), laux[0, 0]


# speedup vs baseline: 1.0249x; 1.0249x over previous
import functools

import jax
import jax.numpy as jnp
from jax import lax
from jax.experimental import pallas as pl
from jax.experimental.pallas import tpu as pltpu
from jax.experimental.pallas import tpu_sc as plsc

S, B, D = 2048, 2, 1024
E, K, C = 64, 2, 256
F_EXP = 512
F_FFN = 2048
T = S * B
KT = K * T
EC = E * C
Y_ROWS = EC + 256
TILE = 256
N_TILES = T // TILE

_f32 = jnp.float32
_i32 = jnp.int32


def _stage1_body(x_ref, res_ref, fw1_ref, fb1_ref, fw2_ref, fb2_ref,
                 lng_ref, lnb_ref, rw_ref,
                 xr_ref, xn_ref, meta_ref, stats_ref):
    i = pl.program_id(0)
    xt = x_ref[...]
    h = jnp.maximum(
        jnp.dot(xt, fw1_ref[...], preferred_element_type=_f32) + fb1_ref[...],
        0.0)
    ff = jnp.dot(h, fw2_ref[...], preferred_element_type=_f32) + fb2_ref[...]
    res_t = jnp.swapaxes(res_ref[...], 0, 1).reshape(TILE, D)
    xr = ff + res_t
    xr_ref[...] = xr
    m = jnp.mean(xr, axis=1, keepdims=True)
    d = xr - m
    v = jnp.mean(d * d, axis=1, keepdims=True)
    xn = d / jnp.sqrt(v + 1e-5) * lng_ref[...] + lnb_ref[...]
    xn_ref[...] = xn
    logits = jnp.dot(xn, rw_ref[...], preferred_element_type=_f32)
    mx = jnp.max(logits, axis=1, keepdims=True)
    ex = jnp.exp(logits - mx)
    p = ex / jnp.sum(ex, axis=1, keepdims=True)
    io = lax.broadcasted_iota(_i32, (TILE, E), 1)
    m0 = jnp.max(p, axis=1, keepdims=True)
    i0 = jnp.min(jnp.where(p == m0, io, E), axis=1, keepdims=True)
    mask0 = io == i0
    p2 = jnp.where(mask0, -1.0, p)
    m1 = jnp.max(p2, axis=1, keepdims=True)
    i1 = jnp.min(jnp.where(p2 == m1, io, E), axis=1, keepdims=True)
    den = m0 + m1 + 1e-9
    g0 = m0 / den
    g1 = m1 / den
    meta_ref[...] = jnp.concatenate(
        [i0.astype(_f32), i1.astype(_f32), g0, g1,
         jnp.zeros((TILE, 4), _f32)], axis=1)
    srow = jnp.concatenate(
        [jnp.sum(p, axis=0, keepdims=True),
         jnp.sum(mask0.astype(_f32), axis=0, keepdims=True)], axis=1)

    @pl.when(i == 0)
    def _():
        stats_ref[...] = jnp.zeros_like(stats_ref)

    stats_ref[...] += srow


def _stage1(x2d, res_bsd, ffn_w1, ffn_b1, ffn_w2, ffn_b2, ln_g, ln_b, router_w):
    tok = lambda i: (i, 0)
    fixed = lambda i: (0, 0)
    return pl.pallas_call(
        _stage1_body,
        grid=(N_TILES,),
        in_specs=[
            pl.BlockSpec((TILE, D), tok),
            pl.BlockSpec((B, TILE // B, D), lambda i: (0, i, 0)),
            pl.BlockSpec((D, F_FFN), fixed),
            pl.BlockSpec((1, F_FFN), fixed),
            pl.BlockSpec((F_FFN, D), fixed),
            pl.BlockSpec((1, D), fixed),
            pl.BlockSpec((1, D), fixed),
            pl.BlockSpec((1, D), fixed),
            pl.BlockSpec((D, E), fixed),
        ],
        out_specs=[
            pl.BlockSpec((TILE, D), tok),
            pl.BlockSpec((TILE, D), tok),
            pl.BlockSpec((TILE, 8), tok),
            pl.BlockSpec((1, 2 * E), fixed),
        ],
        out_shape=[
            jax.ShapeDtypeStruct((T, D), _f32),
            jax.ShapeDtypeStruct((T, D), _f32),
            jax.ShapeDtypeStruct((T, 8), _f32),
            jax.ShapeDtypeStruct((1, 2 * E), _f32),
        ],
    )(x2d, res_bsd, ffn_w1, ffn_b1, ffn_w2, ffn_b2, ln_g, ln_b, router_w)


_RB = 512
_NRB = KT // _RB


def _stage2_body(meta_ref, stats_ref,
                 slot_tok_ref, gate_t_ref, sl_ref, laux_ref,
                 m1_acc, m1l_acc, m2_acc):
    ior = lax.broadcasted_iota(_i32, (_RB, _RB), 0)
    ioc = lax.broadcasted_iota(_i32, (_RB, _RB), 1)
    tri = (ior >= ioc).astype(_f32)
    ioE = lax.broadcasted_iota(_i32, (_RB, E), 1).astype(_f32)
    ioC = lax.broadcasted_iota(_i32, (_RB, C), 1).astype(_f32)
    m1_acc[...] = jnp.zeros_like(m1_acc)
    m1l_acc[...] = jnp.zeros_like(m1l_acc)
    m2_acc[...] = jnp.zeros_like(m2_acc)

    def body(blk, offset):
        k = blk // (_NRB // 2)
        rowoff = (blk % (_NRB // 2)) * _RB
        mb = meta_ref[pl.ds(rowoff, _RB), :]
        e_col = jnp.where(k == 0, mb[:, 0:1], mb[:, 1:2])
        g_col = jnp.where(k == 0, mb[:, 2:3], mb[:, 3:4])
        ohb = (e_col == ioE).astype(_f32)
        cin = jnp.dot(tri, ohb, preferred_element_type=_f32) + offset
        pos = jnp.sum(cin * ohb, axis=1, keepdims=True) - 1.0
        keep = pos < C
        pohb = (pos == ioC).astype(_f32)
        tv = lax.broadcasted_iota(_i32, (_RB, 1), 0).astype(_f32) + rowoff
        thi = jnp.floor(tv * (1.0 / 64.0))
        tlo = tv - thi * 64.0
        cdim = (((0,), (0,)), ((), ()))
        m1_acc[...] += lax.dot_general(ohb * (thi + 1.0), pohb, cdim,
                                       preferred_element_type=_f32)
        m1l_acc[...] += lax.dot_general(ohb * (tlo + 1.0), pohb, cdim,
                                        preferred_element_type=_f32)
        m2_acc[...] += lax.dot_general(pohb, ohb * g_col, cdim,
                                       preferred_element_type=_f32)
        slf = jnp.where(keep, e_col * C + pos, float(EC))
        sl_ref[pl.ds(blk * _RB, _RB), :] = slf.astype(_i32)
        return cin[_RB - 1:_RB, :]

    lax.fori_loop(0, _NRB, body, jnp.zeros((1, E), _f32))

    occ = m1_acc[...] >= 0.5
    tok = (m1_acc[...] - 1.0) * 64.0 + (m1l_acc[...] - 1.0)
    fb_e = lax.broadcasted_iota(_i32, (E, C), 0)
    fb_c = lax.broadcasted_iota(_i32, (E, C), 1)
    fallback = (fb_e * C + fb_c) & (T - 1)
    slot_tok_ref[...] = jnp.where(occ, tok.astype(_i32), fallback)
    gate_t_ref[...] = m2_acc[...]
    me = stats_ref[0:1, 0:E] * (1.0 / T)
    ce = stats_ref[0:1, E:2 * E] * (1.0 / T)
    laux_ref[...] = jnp.sum(me * ce, axis=1, keepdims=True) * float(E)


def _stage2(meta, stats):
    return pl.pallas_call(
        _stage2_body,
        out_shape=[
            jax.ShapeDtypeStruct((E, C), _i32),
            jax.ShapeDtypeStruct((C, E), _f32),
            jax.ShapeDtypeStruct((KT, 1), _i32),
            jax.ShapeDtypeStruct((1, 1), _f32),
        ],
        scratch_shapes=[
            pltpu.VMEM((E, C), _f32),
            pltpu.VMEM((E, C), _f32),
            pltpu.VMEM((C, E), _f32),
        ],
    )(meta, stats)


_NC, _NS = 2, 16
_NW = _NC * _NS
_CH = 32


def _sc_gather(table, idx):
    n, d = table.shape
    r = idx.shape[0]
    rpw = r // _NW
    nch = rpw // _CH
    mesh = plsc.VectorSubcoreMesh(core_axis_name="c", subcore_axis_name="s")

    dt = table.dtype

    def body(table_ref, idx_ref, out_ref, idx_v, buf0, buf1, sem0, sem1):
        wid = lax.axis_index("s") * _NC + lax.axis_index("c")
        base = wid * rpw
        pltpu.sync_copy(idx_ref.at[pl.ds(base, rpw)], idx_v)
        bufs = (buf0, buf1)
        sems = (sem0, sem1)
        cps = [None, None]
        for j in range(nch):
            bsel = j & 1
            cps[bsel] = pltpu.async_copy(
                table_ref.at[idx_v.at[pl.ds(j * _CH, _CH)]], bufs[bsel],
                sems[bsel])
            if j >= 1:
                cps[1 - bsel].wait()
                pltpu.sync_copy(bufs[1 - bsel],
                                out_ref.at[pl.ds(base + (j - 1) * _CH, _CH)])
        last = (nch - 1) & 1
        cps[last].wait()
        pltpu.sync_copy(bufs[last],
                        out_ref.at[pl.ds(base + (nch - 1) * _CH, _CH)])

    return pl.kernel(
        body,
        out_type=jax.ShapeDtypeStruct((r, d), dt),
        mesh=mesh,
        scratch_types=[
            pltpu.VMEM((rpw,), _i32),
            pltpu.VMEM((_CH, d), dt),
            pltpu.VMEM((_CH, d), dt),
            pltpu.SemaphoreType.DMA,
            pltpu.SemaphoreType.DMA,
        ],
    )(table, idx)


_EH = E // 2


def _expert_block(buf_ref, w1_ref, b1_ref, w2_ref, b2_ref, gate_ref, eg):
    xb = buf_ref[...].astype(jnp.bfloat16)
    w1b = w1_ref[0].astype(jnp.bfloat16)
    h = jnp.maximum(
        jnp.dot(xb, w1b, preferred_element_type=_f32) + b1_ref[0], 0.0)
    w2b = w2_ref[0].astype(jnp.bfloat16)
    y = jnp.dot(h.astype(jnp.bfloat16), w2b,
                preferred_element_type=_f32) + b2_ref[0]
    ioe = lax.broadcasted_iota(_i32, (C, E), 1)
    gcol = jnp.sum(jnp.where(ioe == eg, gate_ref[...], 0.0),
                   axis=1, keepdims=True)
    return y * gcol


def _stage4a_body(buf_ref, w1_ref, b1_ref, w2_ref, b2_ref, gate_ref, y_ref):
    e = pl.program_id(0)
    y_ref[...] = _expert_block(buf_ref, w1_ref, b1_ref, w2_ref, b2_ref,
                               gate_ref, e)


def _stage4b_body(yprev_ref, buf_ref, w1_ref, b1_ref, w2_ref, b2_ref,
                  gate_ref, y_ref):
    del yprev_ref
    e = pl.program_id(0)

    @pl.when(e < _EH)
    def _():
        y_ref[...] = _expert_block(buf_ref, w1_ref, b1_ref, w2_ref, b2_ref,
                                   gate_ref, e + _EH)

    @pl.when(e == _EH)
    def _():
        y_ref[...] = jnp.zeros_like(y_ref)


def _expert_specs(off, clamp):
    em = lambda e: (jnp.minimum(e + off, clamp), 0)
    em3 = lambda e: (jnp.minimum(e + off, clamp), 0, 0)
    fixed = lambda e: (0, 0)
    return [
        pl.BlockSpec((C, D), lambda e: (jnp.minimum(e, _EH - 1), 0)),
        pl.BlockSpec((1, D, F_EXP), em3),
        pl.BlockSpec((1, 1, F_EXP), em3),
        pl.BlockSpec((1, F_EXP, D), em3),
        pl.BlockSpec((1, 1, D), em3),
        pl.BlockSpec((C, E), fixed),
    ]


def _stage4(buf_a, buf_b, w1, b1, w2, b2, gate_t):
    b1r = b1.reshape(E, 1, F_EXP)
    b2r = b2.reshape(E, 1, D)
    y1 = pl.pallas_call(
        _stage4a_body,
        grid=(_EH,),
        in_specs=_expert_specs(0, E - 1),
        out_specs=pl.BlockSpec((C, D), lambda e: (e, 0)),
        out_shape=jax.ShapeDtypeStruct((Y_ROWS, D), _f32),
    )(buf_a, w1, b1r, w2, b2r, gate_t)
    return pl.pallas_call(
        _stage4b_body,
        grid=(_EH + 1,),
        in_specs=[pl.BlockSpec(memory_space=pl.ANY)]
        + _expert_specs(_EH, E - 1),
        out_specs=pl.BlockSpec((C, D), lambda e: (e + _EH, 0)),
        out_shape=jax.ShapeDtypeStruct((Y_ROWS, D), _f32),
        input_output_aliases={0: 0},
    )(y1, buf_b, w1, b1r, w2, b2r, gate_t)


def _combine_body(ya_ref, yb_ref, xr_ref, out_ref):
    out_ref[...] = ya_ref[...] + yb_ref[...] + xr_ref[...]


def _combine(g8, xr):
    return pl.pallas_call(
        _combine_body,
        grid=(N_TILES,),
        in_specs=[
            pl.BlockSpec((TILE, D), lambda i: (i, 0)),
            pl.BlockSpec((TILE, D), lambda i: (i + N_TILES, 0)),
            pl.BlockSpec((TILE, D), lambda i: (i, 0)),
        ],
        out_specs=pl.BlockSpec((TILE, D), lambda i: (i, 0)),
        out_shape=jax.ShapeDtypeStruct((T, D), _f32),
    )(g8, g8, xr)


def kernel(x, residual, router_w, w1, b1, w2, b2,
           ffn_w1, ffn_b1, ffn_w2, ffn_b2, ln_g, ln_b):
    x2d = x.reshape(T, D)
    xr, xn, meta, stats = _stage1(
        x2d, residual, ffn_w1, ffn_b1.reshape(1, F_FFN), ffn_w2,
        ffn_b2.reshape(1, D), ln_g.reshape(1, D), ln_b.reshape(1, D),
        router_w)
    slot_tok, gate_t, sl, laux = _stage2(meta, stats)
    slot_flat = slot_tok.reshape(-1)
    buf_a = _sc_gather(xn, slot_flat[:EC // 2])
    buf_b = _sc_gather(xn, slot_flat[EC // 2:])
    y = _stage4(buf_a, buf_b, w1, b1, w2, b2, gate_t)
    g8 = _sc_gather(y, sl.reshape(-1))
    out2d = _combine(g8, xr)
    return out2d.reshape(S, B, D), laux[0, 0]
